# baseline (speedup 1.0000x reference)
import functools

import jax
import jax.numpy as jnp
from jax import lax
from jax.experimental import pallas as pl
from jax.experimental.pallas import tpu as pltpu
from jax.experimental.pallas import tpu_sc as plsc

N = 10000
E = 320000
HID = 128
LAYERS = 3

NCORES = 2
NSUB = 16
NWORK = NCORES * NSUB
CHUNK = 128
NCHUNK = 80
HALFC = NCHUNK // 2
EPW = NCHUNK * CHUNK
EPAD = NWORK * EPW
ACC_ROWS = 10240
PAD_ROWS = ACC_ROWS - N
SLAB = ACC_ROWS // NSUB

ROW_BLK = 400
NBLK = N // ROW_BLK


def _agg_body(h_hbm, src_hbm, dst_hbm, zero_hbm, out_hbm,
              srcv, dstv, rows, acc, gsem, ssem):
    cid = lax.axis_index("c")
    sid = lax.axis_index("s")
    wid = sid * NCORES + cid

    pltpu.sync_copy(zero_hbm, acc.at[pl.ds(sid * SLAB, SLAB)])

    for hh in range(2):
        pltpu.sync_copy(src_hbm.at[wid, pl.ds(hh * HALFC, HALFC)], srcv)
        pltpu.sync_copy(dst_hbm.at[wid, pl.ds(hh * HALFC, HALFC)], dstv)
        if hh == 0:
            plsc.subcore_barrier()
        pltpu.async_copy(h_hbm.at[srcv.at[0]], rows.at[0], gsem.at[0])

        def step(j, _):
            b = lax.rem(j, 2)
            nb = 1 - b
            pltpu.make_async_copy(h_hbm.at[srcv.at[j]], rows.at[b],
                                  gsem.at[b]).wait()
            pltpu.async_copy(rows.at[b], acc.at[dstv.at[j]],
                             ssem.at[b], add=True)

            @pl.when(j + 1 < HALFC)
            def _():
                @pl.when(j >= 1)
                def _():
                    pltpu.make_async_copy(rows.at[nb],
                                          acc.at[dstv.at[j - 1]],
                                          ssem.at[nb]).wait()
                pltpu.async_copy(h_hbm.at[srcv.at[j + 1]], rows.at[nb],
                                 gsem.at[nb])

            return 0

        lax.fori_loop(0, HALFC, step, 0)
        pltpu.make_async_copy(rows.at[(HALFC - 1) % 2],
                              acc.at[dstv.at[HALFC - 1]],
                              ssem.at[(HALFC - 1) % 2]).wait()
        pltpu.make_async_copy(rows.at[HALFC % 2],
                              acc.at[dstv.at[HALFC - 2]],
                              ssem.at[HALFC % 2]).wait()

    plsc.subcore_barrier()
    pltpu.sync_copy(acc.at[pl.ds(sid * SLAB, SLAB)],
                    out_hbm.at[cid, pl.ds(sid * SLAB, SLAB)])


_agg = functools.partial(
    pl.kernel,
    out_type=jax.ShapeDtypeStruct((NCORES, ACC_ROWS, HID), jnp.float32),
    mesh=plsc.VectorSubcoreMesh(core_axis_name="c", subcore_axis_name="s",
                                num_cores=NCORES, num_subcores=NSUB),
    scratch_types=[
        pltpu.VMEM((HALFC, CHUNK), jnp.int32),
        pltpu.VMEM((HALFC, CHUNK), jnp.int32),
        pltpu.VMEM((2, CHUNK, HID), jnp.float32),
        pltpu.VMEM_SHARED((ACC_ROWS, HID), jnp.float32),
        pltpu.SemaphoreType.DMA((2,)),
        pltpu.SemaphoreType.DMA((2,)),
    ],
)(_agg_body)


def _mlp_body(h_ref, p0_ref, p1_ref, w1_ref, b1_ref, w2_ref, b2_ref,
              wo_ref, oin_ref, h_out, o_out):
    m = h_ref[...] + p0_ref[0] + p1_ref[0]
    t = jnp.maximum(jnp.dot(m, w1_ref[...],
                            preferred_element_type=jnp.float32) + b1_ref[...], 0.0)
    hn = jnp.maximum(jnp.dot(t, w2_ref[...],
                             preferred_element_type=jnp.float32) + b2_ref[...], 0.0)
    h_out[...] = hn
    o_out[...] = oin_ref[...] + jnp.dot(hn, wo_ref[...],
                                        preferred_element_type=jnp.float32)


def _mlp1_body(x_ref, p0_ref, p1_ref, w1_ref, b1_ref, w2_ref, b2_ref,
               wo0_ref, wo1_ref, ob_ref, h_out, o_out):
    m = x_ref[...] + p0_ref[0] + p1_ref[0]
    t = jnp.maximum(jnp.dot(m, w1_ref[...],
                            preferred_element_type=jnp.float32) + b1_ref[...], 0.0)
    hn = jnp.maximum(jnp.dot(t, w2_ref[...],
                             preferred_element_type=jnp.float32) + b2_ref[...], 0.0)
    h_out[...] = hn
    o_out[...] = (jnp.dot(x_ref[...], wo0_ref[...],
                          preferred_element_type=jnp.float32) + ob_ref[...]
                  + jnp.dot(hn, wo1_ref[...],
                            preferred_element_type=jnp.float32))


_row_spec = pl.BlockSpec((ROW_BLK, HID), lambda i: (i, 0))
_part_spec0 = pl.BlockSpec((1, ROW_BLK, HID), lambda i: (0, i, 0))
_part_spec1 = pl.BlockSpec((1, ROW_BLK, HID), lambda i: (1, i, 0))
_w_spec = pl.BlockSpec((HID, HID), lambda i: (0, 0))
_b_spec = pl.BlockSpec((1, HID), lambda i: (0, 0))
_out2 = [jax.ShapeDtypeStruct((N, HID), jnp.float32),
         jax.ShapeDtypeStruct((N, HID), jnp.float32)]

_mlp = pl.pallas_call(
    _mlp_body,
    grid=(NBLK,),
    in_specs=[_row_spec, _part_spec0, _part_spec1, _w_spec, _b_spec,
              _w_spec, _b_spec, _w_spec, _row_spec],
    out_specs=[_row_spec, _row_spec],
    out_shape=_out2,
)

_mlp1 = pl.pallas_call(
    _mlp1_body,
    grid=(NBLK,),
    in_specs=[_row_spec, _part_spec0, _part_spec1, _w_spec, _b_spec,
              _w_spec, _b_spec, _w_spec, _w_spec, _b_spec],
    out_specs=[_row_spec, _row_spec],
    out_shape=_out2,
)


def kernel(x, edge_index, gin_w1, gin_b1, gin_w2, gin_b2, out_w, out_b):
    src = edge_index[0].astype(jnp.int32)
    dst = edge_index[1].astype(jnp.int32)
    npad = EPAD - E
    pad_src = jnp.arange(npad, dtype=jnp.int32) % N
    pad_dst = N + jnp.arange(npad, dtype=jnp.int32) % PAD_ROWS
    src_r = jnp.concatenate([src, pad_src]).reshape(NWORK, NCHUNK, CHUNK)
    dst_r = jnp.concatenate([dst, pad_dst]).reshape(NWORK, NCHUNK, CHUNK)
    zeros = jnp.zeros((SLAB, HID), jnp.float32)

    b1 = gin_b1.reshape(LAYERS, 1, HID)
    b2 = gin_b2.reshape(LAYERS, 1, HID)
    ob = out_b.reshape(1, HID)
    wo = out_w.reshape(LAYERS + 1, HID, HID)

    h = x
    outp = None
    for l in range(LAYERS):
        parts = _agg(h, src_r, dst_r, zeros)
        if l == 0:
            h, outp = _mlp1(x, parts, parts, gin_w1[0], b1[0], gin_w2[0],
                            b2[0], wo[0], wo[1], ob)
        else:
            h, outp = _mlp(h, parts, parts, gin_w1[l], b1[l], gin_w2[l],
                           b2[l], wo[l + 1], outp)
    return outp


# speedup vs baseline: 1.1229x; 1.1229x over previous
import functools

import jax
import jax.numpy as jnp
from jax import lax
from jax.experimental import pallas as pl
from jax.experimental.pallas import tpu as pltpu
from jax.experimental.pallas import tpu_sc as plsc

N = 10000
E = 320000
HID = 128
LAYERS = 3

NCORES = 2
NSUB = 16
NWORK = NCORES * NSUB
CHUNK = 64
NCHUNK = 160
NBUF = 4
NSTG = 4
STG = NCHUNK // NSTG
EPW = NCHUNK * CHUNK
EPAD = NWORK * EPW
ACC_ROWS = 10240
PAD_ROWS = ACC_ROWS - N
SLAB = ACC_ROWS // NSUB

ROW_BLK = 400
NBLK = N // ROW_BLK


def _agg_body(h_hbm, src_hbm, dst_hbm, zero_hbm, out_hbm,
              srcv, dstv, rows, acc, gsem, ssem):
    cid = lax.axis_index("c")
    sid = lax.axis_index("s")
    wid = sid * NCORES + cid

    pltpu.sync_copy(zero_hbm, acc.at[pl.ds(sid * SLAB, SLAB)])

    for st in range(NSTG):
        pltpu.sync_copy(src_hbm.at[wid, pl.ds(st * STG, STG)], srcv)
        pltpu.sync_copy(dst_hbm.at[wid, pl.ds(st * STG, STG)], dstv)
        if st == 0:
            plsc.subcore_barrier()
        for p in range(NBUF - 1):
            pltpu.async_copy(h_hbm.at[srcv.at[p]], rows.at[p], gsem.at[p])

        def step(j, _):
            b = lax.rem(j, NBUF)
            pltpu.make_async_copy(h_hbm.at[srcv.at[j]], rows.at[b],
                                  gsem.at[b]).wait()
            pltpu.async_copy(rows.at[b], acc.at[dstv.at[j]],
                             ssem.at[b], add=True)

            @pl.when(j + NBUF - 1 < STG)
            def _():
                nb = lax.rem(j + NBUF - 1, NBUF)

                @pl.when(j >= 1)
                def _():
                    pltpu.make_async_copy(rows.at[nb],
                                          acc.at[dstv.at[j - 1]],
                                          ssem.at[nb]).wait()
                pltpu.async_copy(h_hbm.at[srcv.at[j + NBUF - 1]],
                                 rows.at[nb], gsem.at[nb])

            return 0

        lax.fori_loop(0, STG, step, 0)
        for p in range(NBUF):
            pltpu.make_async_copy(rows.at[p], acc.at[dstv.at[0]],
                                  ssem.at[p]).wait()

    plsc.subcore_barrier()
    pltpu.sync_copy(acc.at[pl.ds(sid * SLAB, SLAB)],
                    out_hbm.at[cid, pl.ds(sid * SLAB, SLAB)])


_agg = functools.partial(
    pl.kernel,
    out_type=jax.ShapeDtypeStruct((NCORES, ACC_ROWS, HID), jnp.float32),
    mesh=plsc.VectorSubcoreMesh(core_axis_name="c", subcore_axis_name="s",
                                num_cores=NCORES, num_subcores=NSUB),
    scratch_types=[
        pltpu.VMEM((STG, CHUNK), jnp.int32),
        pltpu.VMEM((STG, CHUNK), jnp.int32),
        pltpu.VMEM((NBUF, CHUNK, HID), jnp.float32),
        pltpu.VMEM_SHARED((ACC_ROWS, HID), jnp.float32),
        pltpu.SemaphoreType.DMA((NBUF,)),
        pltpu.SemaphoreType.DMA((NBUF,)),
    ],
)(_agg_body)


def _mlp_body(h_ref, p0_ref, p1_ref, w1_ref, b1_ref, w2_ref, b2_ref,
              wo_ref, oin_ref, h_out, o_out):
    m = h_ref[...] + p0_ref[0] + p1_ref[0]
    t = jnp.maximum(jnp.dot(m, w1_ref[...],
                            preferred_element_type=jnp.float32) + b1_ref[...], 0.0)
    hn = jnp.maximum(jnp.dot(t, w2_ref[...],
                             preferred_element_type=jnp.float32) + b2_ref[...], 0.0)
    h_out[...] = hn
    o_out[...] = oin_ref[...] + jnp.dot(hn, wo_ref[...],
                                        preferred_element_type=jnp.float32)


def _mlp1_body(x_ref, p0_ref, p1_ref, w1_ref, b1_ref, w2_ref, b2_ref,
               wo0_ref, wo1_ref, ob_ref, h_out, o_out):
    m = x_ref[...] + p0_ref[0] + p1_ref[0]
    t = jnp.maximum(jnp.dot(m, w1_ref[...],
                            preferred_element_type=jnp.float32) + b1_ref[...], 0.0)
    hn = jnp.maximum(jnp.dot(t, w2_ref[...],
                             preferred_element_type=jnp.float32) + b2_ref[...], 0.0)
    h_out[...] = hn
    o_out[...] = (jnp.dot(x_ref[...], wo0_ref[...],
                          preferred_element_type=jnp.float32) + ob_ref[...]
                  + jnp.dot(hn, wo1_ref[...],
                            preferred_element_type=jnp.float32))


_row_spec = pl.BlockSpec((ROW_BLK, HID), lambda i: (i, 0))
_part_spec0 = pl.BlockSpec((1, ROW_BLK, HID), lambda i: (0, i, 0))
_part_spec1 = pl.BlockSpec((1, ROW_BLK, HID), lambda i: (1, i, 0))
_w_spec = pl.BlockSpec((HID, HID), lambda i: (0, 0))
_b_spec = pl.BlockSpec((1, HID), lambda i: (0, 0))
_out2 = [jax.ShapeDtypeStruct((N, HID), jnp.float32),
         jax.ShapeDtypeStruct((N, HID), jnp.float32)]

_mlp = pl.pallas_call(
    _mlp_body,
    grid=(NBLK,),
    in_specs=[_row_spec, _part_spec0, _part_spec1, _w_spec, _b_spec,
              _w_spec, _b_spec, _w_spec, _row_spec],
    out_specs=[_row_spec, _row_spec],
    out_shape=_out2,
)

_mlp1 = pl.pallas_call(
    _mlp1_body,
    grid=(NBLK,),
    in_specs=[_row_spec, _part_spec0, _part_spec1, _w_spec, _b_spec,
              _w_spec, _b_spec, _w_spec, _w_spec, _b_spec],
    out_specs=[_row_spec, _row_spec],
    out_shape=_out2,
)


def kernel(x, edge_index, gin_w1, gin_b1, gin_w2, gin_b2, out_w, out_b):
    src = edge_index[0].astype(jnp.int32)
    dst = edge_index[1].astype(jnp.int32)
    npad = EPAD - E
    pad_src = jnp.arange(npad, dtype=jnp.int32) % N
    pad_dst = N + jnp.arange(npad, dtype=jnp.int32) % PAD_ROWS
    src_r = jnp.concatenate([src, pad_src]).reshape(NWORK, NCHUNK, CHUNK)
    dst_r = jnp.concatenate([dst, pad_dst]).reshape(NWORK, NCHUNK, CHUNK)
    zeros = jnp.zeros((SLAB, HID), jnp.float32)

    b1 = gin_b1.reshape(LAYERS, 1, HID)
    b2 = gin_b2.reshape(LAYERS, 1, HID)
    ob = out_b.reshape(1, HID)
    wo = out_w.reshape(LAYERS + 1, HID, HID)

    h = x
    outp = None
    for l in range(LAYERS):
        parts = _agg(h, src_r, dst_r, zeros)
        if l == 0:
            h, outp = _mlp1(x, parts, parts, gin_w1[0], b1[0], gin_w2[0],
                            b2[0], wo[0], wo[1], ob)
        else:
            h, outp = _mlp(h, parts, parts, gin_w1[l], b1[l], gin_w2[l],
                           b2[l], wo[l + 1], outp)
    return outp
